# Initial kernel scaffold; baseline (speedup 1.0000x reference)
import jax
import jax.numpy as jnp
from jax.experimental import pallas as pl


def kernel(padded_action, action_type, emb, W1, b1, W2, b2):
    raise NotImplementedError("write your pallas kernel here")


# speedup vs baseline: 2.5465x; 2.5465x over previous
import jax
import jax.numpy as jnp
from jax.experimental import pallas as pl
from jax.experimental.pallas import tpu as pltpu

_B, _T, _A = 32, 2048, 64
_NS, _ED, _H, _D = 32, 64, 512, 1024
_TT = 2048


def _c_kernel(at_ref, emb_ref, w1b_ref, b1_ref, c_ref):
    at = at_ref[...]
    niota = jax.lax.broadcasted_iota(jnp.int32, (_NS, _B), 0)
    onehot_t = (niota == at).astype(jnp.float32)
    g = jax.lax.dot_general(onehot_t, emb_ref[...],
                            (((0,), (0,)), ((), ())),
                            preferred_element_type=jnp.float32)
    c_ref[...] = jnp.dot(g, w1b_ref[...],
                         preferred_element_type=jnp.float32) + b1_ref[...]


def _mlp_kernel(x_ref, c_ref, w1a_ref, w2_ref, b2_ref, o_ref):
    x = x_ref[...].astype(jnp.bfloat16)
    h = jnp.dot(x, w1a_ref[...], preferred_element_type=jnp.float32)
    h = jnp.maximum(h + c_ref[0], 0.0).astype(jnp.bfloat16)
    o_ref[...] = jnp.dot(h, w2_ref[...],
                         preferred_element_type=jnp.float32) + b2_ref[...]


def kernel(padded_action, action_type, emb, W1, b1, W2, b2):
    x = padded_action.reshape(_B * _T, _A)
    at2 = action_type.reshape(1, _B).astype(jnp.int32)
    w1a = W1[:_A].astype(jnp.bfloat16)
    w1b = W1[_A:]
    b1r = b1.reshape(1, _H)
    w2 = W2.astype(jnp.bfloat16)
    b2r = b2.reshape(1, _D)

    c = pl.pallas_call(
        _c_kernel,
        out_shape=jax.ShapeDtypeStruct((_B, _H), jnp.float32),
        in_specs=[
            pl.BlockSpec((1, _B), lambda: (0, 0)),
            pl.BlockSpec((_NS, _ED), lambda: (0, 0)),
            pl.BlockSpec((_ED, _H), lambda: (0, 0)),
            pl.BlockSpec((1, _H), lambda: (0, 0)),
        ],
        out_specs=pl.BlockSpec((_B, _H), lambda: (0, 0)),
    )(at2, emb, w1b, b1r)

    c3 = c.reshape(_B, 1, _H)
    steps_per_batch = _T // _TT
    grid = (_B * _T) // _TT
    out = pl.pallas_call(
        _mlp_kernel,
        grid=(grid,),
        out_shape=jax.ShapeDtypeStruct((_B * _T, _D), jnp.float32),
        in_specs=[
            pl.BlockSpec((_TT, _A), lambda i: (i, 0)),
            pl.BlockSpec((1, 1, _H), lambda i: (i // steps_per_batch, 0, 0)),
            pl.BlockSpec((_A, _H), lambda i: (0, 0)),
            pl.BlockSpec((_H, _D), lambda i: (0, 0)),
            pl.BlockSpec((1, _D), lambda i: (0, 0)),
        ],
        out_specs=pl.BlockSpec((_TT, _D), lambda i: (i, 0)),
        compiler_params=pltpu.CompilerParams(
            dimension_semantics=("parallel",)),
    )(x, c3, w1a, w2, b2r)
    return out.reshape(_B, _T, _D)


# speedup vs baseline: 2.5483x; 1.0007x over previous
import jax
import jax.numpy as jnp
from jax.experimental import pallas as pl
from jax.experimental.pallas import tpu as pltpu

_B, _T, _A = 32, 2048, 64
_NS, _ED, _H, _D = 32, 64, 512, 1024
_TT = 2048


def _c_kernel(at_ref, emb_ref, w1b_ref, b1_ref, c_ref):
    at = at_ref[...]
    niota = jax.lax.broadcasted_iota(jnp.int32, (_NS, _B), 0)
    onehot_t = (niota == at).astype(jnp.float32)
    g = jax.lax.dot_general(onehot_t, emb_ref[...],
                            (((0,), (0,)), ((), ())),
                            preferred_element_type=jnp.float32)
    c_ref[...] = jnp.dot(g, w1b_ref[...],
                         preferred_element_type=jnp.float32) + b1_ref[...]


def _mlp_kernel(x_ref, c_ref, w1a_ref, w2_ref, b2_ref, o_ref):
    x = x_ref[0].astype(jnp.bfloat16)
    h = jnp.dot(x, w1a_ref[...], preferred_element_type=jnp.float32)
    h = jnp.maximum(h + c_ref[0], 0.0).astype(jnp.bfloat16)
    o_ref[0] = jnp.dot(h, w2_ref[...],
                       preferred_element_type=jnp.float32) + b2_ref[...]


def kernel(padded_action, action_type, emb, W1, b1, W2, b2):
    at2 = action_type.reshape(1, _B).astype(jnp.int32)
    w1a = W1[:_A].astype(jnp.bfloat16)
    w1b = W1[_A:]
    b1r = b1.reshape(1, _H)
    w2 = W2.astype(jnp.bfloat16)
    b2r = b2.reshape(1, _D)

    c = pl.pallas_call(
        _c_kernel,
        out_shape=jax.ShapeDtypeStruct((_B, _H), jnp.float32),
        in_specs=[
            pl.BlockSpec((1, _B), lambda: (0, 0)),
            pl.BlockSpec((_NS, _ED), lambda: (0, 0)),
            pl.BlockSpec((_ED, _H), lambda: (0, 0)),
            pl.BlockSpec((1, _H), lambda: (0, 0)),
        ],
        out_specs=pl.BlockSpec((_B, _H), lambda: (0, 0)),
    )(at2, emb, w1b, b1r)

    c3 = c.reshape(_B, 1, _H)
    steps_per_batch = _T // _TT
    grid = (_B, steps_per_batch)
    out = pl.pallas_call(
        _mlp_kernel,
        grid=grid,
        out_shape=jax.ShapeDtypeStruct((_B, _T, _D), jnp.float32),
        in_specs=[
            pl.BlockSpec((1, _TT, _A), lambda b, t: (b, t, 0)),
            pl.BlockSpec((1, 1, _H), lambda b, t: (b, 0, 0)),
            pl.BlockSpec((_A, _H), lambda b, t: (0, 0)),
            pl.BlockSpec((_H, _D), lambda b, t: (0, 0)),
            pl.BlockSpec((1, _D), lambda b, t: (0, 0)),
        ],
        out_specs=pl.BlockSpec((1, _TT, _D), lambda b, t: (b, t, 0)),
        compiler_params=pltpu.CompilerParams(
            dimension_semantics=("parallel", "parallel")),
    )(padded_action, c3, w1a, w2, b2r)
    return out
